# baseline (speedup 1.0000x reference)
import functools

import jax
import jax.numpy as jnp
from jax import lax
from jax.experimental import pallas as pl
from jax.experimental.pallas import tpu as pltpu
from jax.experimental.pallas import tpu_sc as plsc

_NC = 2
_NS = 16
_NW = _NC * _NS
_L = 16


@functools.lru_cache(maxsize=None)
def _make_gather(S, B, D, dtype_name):
    dtype = jnp.dtype(dtype_name)
    BB = B // _NW
    mesh = plsc.VectorSubcoreMesh(core_axis_name="c", subcore_axis_name="s")

    def body(idx_hbm, table_hbm, out_hbm, idx_v, widx_v, wbuf, tbuf, gsem):
        wid = lax.axis_index("s") * _NC + lax.axis_index("c")
        b0 = wid * BB
        pltpu.sync_copy(idx_hbm.at[:, pl.ds(b0, BB)], idx_v)

        @pl.loop(0, S)
        def _(s):
            for g in range(BB // _L):
                iv = idx_v[s, pl.ds(g * _L, _L)]
                widx_v[pl.ds(g * _L, _L)] = lax.shift_right_logical(iv, 1)
            pltpu.async_copy(table_hbm.at[widx_v], wbuf, gsem).wait()
            for g in range(BB // _L):
                iv = idx_v[s, pl.ds(g * _L, _L)]
                cb = (iv & jnp.int32(1)) * jnp.int32(D)
                rowv = lax.iota(jnp.int32, _L) + jnp.int32(g * _L)

                @pl.loop(0, D, step=4)
                def _(d):
                    for dd in range(4):
                        v = plsc.load_gather(wbuf, [rowv, cb + (d + dd)])
                        tbuf[d + dd, pl.ds(g * _L, _L)] = v
            pltpu.sync_copy(tbuf, out_hbm.at[s, :, pl.ds(b0, BB)])

    return pl.kernel(
        body,
        out_type=jax.ShapeDtypeStruct((S, D, B), dtype),
        mesh=mesh,
        compiler_params=pltpu.CompilerParams(
            use_tc_tiling_on_sc=True, needs_layout_passes=False),
        scratch_types=[
            pltpu.VMEM((S, BB), jnp.int32),
            pltpu.VMEM((BB,), jnp.int32),
            pltpu.VMEM((BB, 2 * D), dtype),
            pltpu.VMEM((D, BB), dtype),
            pltpu.SemaphoreType.DMA,
        ],
    )


def kernel(indices, table):
    B, S = indices.shape
    V, D = table.shape
    table2 = table.reshape(V // 2, 2 * D)
    idx_t = indices.astype(jnp.int32).T
    out_k = _make_gather(S, B, D, table.dtype.name)(idx_t, table2)
    return jnp.transpose(out_k, (2, 0, 1))


# speedup vs baseline: 1.1327x; 1.1327x over previous
import functools

import jax
import jax.numpy as jnp
from jax import lax
from jax.experimental import pallas as pl
from jax.experimental.pallas import tpu as pltpu
from jax.experimental.pallas import tpu_sc as plsc

_NC = 2
_NS = 16
_NW = _NC * _NS
_L = 16


@functools.lru_cache(maxsize=None)
def _make_gather(S, B, D, dtype_name):
    dtype = jnp.dtype(dtype_name)
    BB = B // _NW
    mesh = plsc.VectorSubcoreMesh(core_axis_name="c", subcore_axis_name="s")

    def body(idx_hbm, table_hbm, out_hbm, idx_v, widx2, wbuf2, tbuf2, gsem, wsem):
        wid = lax.axis_index("s") * _NC + lax.axis_index("c")
        b0 = wid * BB
        pltpu.sync_copy(idx_hbm.at[:, pl.ds(b0, BB)], idx_v)

        def comp_widx(s, h):
            for g in range(BB // _L):
                iv = idx_v[s, pl.ds(g * _L, _L)]
                widx2[h, pl.ds(g * _L, _L)] = lax.shift_right_logical(iv, 1)

        def gdesc(h):
            return pltpu.make_async_copy(
                table_hbm.at[widx2.at[h]], wbuf2.at[h], gsem)

        def wdesc(s, h):
            return pltpu.make_async_copy(
                tbuf2.at[h], out_hbm.at[s, :, pl.ds(b0, BB)], wsem)

        def transpose(s, h):
            for g in range(BB // _L):
                iv = idx_v[s, pl.ds(g * _L, _L)]
                cb = (iv & jnp.int32(1)) * jnp.int32(D)
                rowv = lax.iota(jnp.int32, _L) + jnp.int32(g * _L)
                for d in range(D):
                    v = plsc.load_gather(wbuf2.at[h], [rowv, cb + d])
                    tbuf2[h, d, pl.ds(g * _L, _L)] = v

        comp_widx(0, 0)
        gdesc(0).start()

        @pl.loop(0, S, step=2)
        def _(s0):
            for h in range(2):
                s = s0 + h
                gdesc(h).wait()

                @pl.when(s + 1 < S)
                def _():
                    comp_widx(s + 1, 1 - h)
                    gdesc(1 - h).start()

                @pl.when(s >= 2)
                def _():
                    wdesc(0, h).wait()

                transpose(s, h)
                wdesc(s, h).start()

        for h in range(2):
            wdesc(0, h).wait()

    return pl.kernel(
        body,
        out_type=jax.ShapeDtypeStruct((S, D, B), dtype),
        mesh=mesh,
        compiler_params=pltpu.CompilerParams(
            use_tc_tiling_on_sc=True, needs_layout_passes=False),
        scratch_types=[
            pltpu.VMEM((S, BB), jnp.int32),
            pltpu.VMEM((2, BB), jnp.int32),
            pltpu.VMEM((2, BB, 2 * D), dtype),
            pltpu.VMEM((2, D, BB), dtype),
            pltpu.SemaphoreType.DMA,
            pltpu.SemaphoreType.DMA,
        ],
    )


def kernel(indices, table):
    B, S = indices.shape
    V, D = table.shape
    table2 = table.reshape(V // 2, 2 * D)
    idx_t = indices.astype(jnp.int32).T
    out_k = _make_gather(S, B, D, table.dtype.name)(idx_t, table2)
    return jnp.transpose(out_k, (2, 0, 1))
